# baseline (speedup 1.0000x reference)
import functools

import jax
import jax.numpy as jnp
from jax import lax
from jax.experimental import pallas as pl
from jax.experimental.pallas import tpu as pltpu
from jax.experimental.pallas import tpu_sc as plsc

_V = 1000
_B = 1024
_L = 50
_NC, _NS = 2, 16
_LPC = _L // _NC
_BPS = _B // _NS
_PLANE = _V * _B
_WPS = _LPC * _PLANE // _NS
_ZC = 100000
_NZ = _WPS // _ZC


@functools.partial(
    pl.kernel,
    out_type=jax.ShapeDtypeStruct((_B * _L * _V,), jnp.float32),
    mesh=plsc.VectorSubcoreMesh(core_axis_name="c", subcore_axis_name="s"),
    scratch_types=[
        pltpu.VMEM((_BPS * _L,), jnp.int32),
        pltpu.VMEM((_ZC,), jnp.float32),
        pltpu.VMEM((_LPC, _BPS), jnp.int32),
        pltpu.VMEM((_BPS,), jnp.float32),
        pltpu.SemaphoreType.DMA,
        pltpu.SemaphoreType.DMA,
    ],
    compiler_params=pltpu.CompilerParams(needs_layout_passes=False),
)
def _onehot_sc(idx_hbm, z_hbm, out_hbm, idx_v, zbuf, pos_v, ones_v, semz, sems):
    c = lax.axis_index("c")
    s = lax.axis_index("s")
    b0 = s * _BPS
    zone_base = c * (_LPC * _PLANE) + s * _WPS

    pltpu.sync_copy(idx_hbm.at[pl.ds(b0 * _L, _BPS * _L)], idx_v)
    pltpu.sync_copy(z_hbm, zbuf)

    for j in range(_NZ):
        pltpu.async_copy(
            zbuf, out_hbm.at[pl.ds(zone_base + j * _ZC, _ZC)], semz)

    for g in range(_BPS // 16):
        ones_v[pl.ds(g * 16, 16)] = jnp.full((16,), 1.0, jnp.float32)
    for ll in range(_LPC):
        l = c * _LPC + ll
        for g in range(_BPS // 16):
            bvec = lax.iota(jnp.int32, 16) + (g * 16)
            iv = plsc.load_gather(idx_v, [bvec * _L + l])
            pos_v[ll, pl.ds(g * 16, 16)] = (
                (l * _V + iv) * _B + (b0 + g * 16) + lax.iota(jnp.int32, 16))

    for j in range(_NZ):
        pltpu.make_async_copy(
            zbuf, out_hbm.at[pl.ds(zone_base, _ZC)], semz).wait()
    plsc.subcore_barrier()
    for ll in range(_LPC):
        pltpu.async_copy(ones_v, out_hbm.at[pos_v.at[ll]], sems)
    for ll in range(_LPC):
        pltpu.make_async_copy(
            ones_v, out_hbm.at[pos_v.at[0]], sems).wait()


def kernel(input):
    idx_flat = input.reshape(_B * _L)
    z = jnp.zeros((_ZC,), jnp.float32)
    flat = _onehot_sc(idx_flat, z)
    return flat.reshape(_L, _V, _B).transpose(2, 0, 1)


# speedup vs baseline: 2.4458x; 2.4458x over previous
import functools

import jax
import jax.numpy as jnp
from jax import lax
from jax.experimental import pallas as pl
from jax.experimental.pallas import tpu as pltpu
from jax.experimental.pallas import tpu_sc as plsc

_V = 1000
_B = 1024
_L = 50
_NC, _NS = 2, 16
_LPC = _L // _NC
_BPS = _B // _NS
_PLANE = _V * _B
_WPS = _LPC * _PLANE // _NS
_ZC = 100000
_NZ = _WPS // _ZC


@functools.partial(
    pl.kernel,
    out_type=jax.ShapeDtypeStruct((_B * _L * _V,), jnp.float32),
    mesh=plsc.VectorSubcoreMesh(core_axis_name="c", subcore_axis_name="s"),
    scratch_types=[
        pltpu.VMEM((_BPS * _L,), jnp.int32),
        pltpu.VMEM((_ZC,), jnp.float32),
        pltpu.VMEM((_LPC, _BPS), jnp.int32),
        pltpu.VMEM((_BPS,), jnp.float32),
        pltpu.SemaphoreType.DMA,
        pltpu.SemaphoreType.DMA,
    ],
    compiler_params=pltpu.CompilerParams(needs_layout_passes=False),
)
def _onehot_sc(idx_hbm, z_hbm, out_hbm, idx_v, zbuf, pos_v, ones_v, semz, sems):
    c = lax.axis_index("c")
    s = lax.axis_index("s")
    b0 = s * _BPS
    zone_base = c * (_LPC * _PLANE) + s * _WPS

    pltpu.sync_copy(idx_hbm.at[pl.ds(b0 * _L, _BPS * _L)], idx_v)
    pltpu.sync_copy(z_hbm, zbuf)

    for j in range(_NZ):
        pltpu.async_copy(
            zbuf, out_hbm.at[pl.ds(zone_base + j * _ZC, _ZC)], semz)

    for g in range(_BPS // 16):
        ones_v[pl.ds(g * 16, 16)] = jnp.full((16,), 1.0, jnp.float32)
    for ll in range(_LPC):
        l = c * _LPC + ll
        for g in range(_BPS // 16):
            bvec = lax.iota(jnp.int32, 16) + (b0 + g * 16)
            iv = plsc.load_gather(idx_v, [(bvec - b0) * _L + l])
            pos_v[ll, pl.ds(g * 16, 16)] = (
                l * _PLANE + (iv >> 3) * 8192 + (bvec >> 7) * 1024
                + (iv & 7) * 128 + (bvec & 127))

    for j in range(_NZ):
        pltpu.make_async_copy(
            zbuf, out_hbm.at[pl.ds(zone_base, _ZC)], semz).wait()
    plsc.subcore_barrier()
    for ll in range(_LPC):
        pltpu.async_copy(ones_v, out_hbm.at[pos_v.at[ll]], sems)
    for ll in range(_LPC):
        pltpu.make_async_copy(
            ones_v, out_hbm.at[pos_v.at[0]], sems).wait()


def kernel(input):
    idx_flat = input.reshape(_B * _L)
    z = jnp.zeros((_ZC,), jnp.float32)
    flat = _onehot_sc(idx_flat, z)
    t5 = flat.reshape(_L, _V // 8, _B // 128, 8, 128)
    return t5.transpose(2, 4, 0, 1, 3).reshape(_B, _L, _V)


# speedup vs baseline: 2.4977x; 1.0212x over previous
import functools

import jax
import jax.numpy as jnp
from jax import lax
from jax.experimental import pallas as pl
from jax.experimental.pallas import tpu as pltpu
from jax.experimental.pallas import tpu_sc as plsc

_V = 1000
_B = 1024
_L = 50
_NC, _NS = 2, 16
_LPC = _L // _NC
_BPS = _B // _NS
_PLANE = _V * _B
_WPS = _LPC * _PLANE // _NS
_ZC = 50000
_NZ = _WPS // _ZC


@functools.partial(
    pl.kernel,
    out_type=jax.ShapeDtypeStruct((_B * _L * _V,), jnp.float32),
    mesh=plsc.VectorSubcoreMesh(core_axis_name="c", subcore_axis_name="s"),
    scratch_types=[
        pltpu.VMEM((_BPS * _L,), jnp.int32),
        pltpu.VMEM((_ZC,), jnp.float32),
        pltpu.VMEM((_LPC, _BPS), jnp.int32),
        pltpu.VMEM((_BPS,), jnp.float32),
        pltpu.SemaphoreType.DMA,
        pltpu.SemaphoreType.DMA,
    ],
    compiler_params=pltpu.CompilerParams(needs_layout_passes=False),
)
def _onehot_sc(idx_hbm, z_hbm, out_hbm, idx_v, zbuf, pos_v, ones_v, semz, sems):
    c = lax.axis_index("c")
    s = lax.axis_index("s")
    b0 = s * _BPS
    zone_base = c * (_LPC * _PLANE) + s * _WPS

    pltpu.sync_copy(idx_hbm.at[pl.ds(b0 * _L, _BPS * _L)], idx_v)
    pltpu.sync_copy(z_hbm, zbuf)

    for j in range(_NZ):
        pltpu.async_copy(
            zbuf, out_hbm.at[pl.ds(zone_base + j * _ZC, _ZC)], semz)

    for g in range(_BPS // 16):
        ones_v[pl.ds(g * 16, 16)] = jnp.full((16,), 1.0, jnp.float32)
    for ll in range(_LPC):
        l = c * _LPC + ll
        for g in range(_BPS // 16):
            bvec = lax.iota(jnp.int32, 16) + (b0 + g * 16)
            iv = plsc.load_gather(idx_v, [(bvec - b0) * _L + l])
            pos_v[ll, pl.ds(g * 16, 16)] = (
                l * _PLANE + (iv >> 3) * 8192 + (bvec >> 7) * 1024
                + (iv & 7) * 128 + (bvec & 127))

    for j in range(_NZ):
        pltpu.make_async_copy(
            zbuf, out_hbm.at[pl.ds(zone_base, _ZC)], semz).wait()
    plsc.subcore_barrier()
    for ll in range(_LPC):
        pltpu.async_copy(ones_v, out_hbm.at[pos_v.at[ll]], sems)
    for ll in range(_LPC):
        pltpu.make_async_copy(
            ones_v, out_hbm.at[pos_v.at[0]], sems).wait()


def kernel(input):
    idx_flat = input.reshape(_B * _L)
    z = jnp.zeros((_ZC,), jnp.float32)
    flat = _onehot_sc(idx_flat, z)
    t5 = flat.reshape(_L, _V // 8, _B // 128, 8, 128)
    return t5.transpose(2, 4, 0, 1, 3).reshape(_B, _L, _V)


# speedup vs baseline: 2.5388x; 1.0164x over previous
import functools

import jax
import jax.numpy as jnp
from jax import lax
from jax.experimental import pallas as pl
from jax.experimental.pallas import tpu as pltpu
from jax.experimental.pallas import tpu_sc as plsc

_V = 1000
_B = 1024
_L = 50
_NC, _NS = 2, 16
_LPC = _L // _NC
_BPS = _B // _NS
_PLANE = _V * _B
_WPS = _LPC * _PLANE // _NS
_ZC = 25000
_NZ = _WPS // _ZC


@functools.partial(
    pl.kernel,
    out_type=jax.ShapeDtypeStruct((_B * _L * _V,), jnp.float32),
    mesh=plsc.VectorSubcoreMesh(core_axis_name="c", subcore_axis_name="s"),
    scratch_types=[
        pltpu.VMEM((_BPS * _L,), jnp.int32),
        pltpu.VMEM((_ZC,), jnp.float32),
        pltpu.VMEM((_LPC, _BPS), jnp.int32),
        pltpu.VMEM((_BPS,), jnp.float32),
        pltpu.SemaphoreType.DMA,
        pltpu.SemaphoreType.DMA,
    ],
    compiler_params=pltpu.CompilerParams(needs_layout_passes=False),
)
def _onehot_sc(idx_hbm, z_hbm, out_hbm, idx_v, zbuf, pos_v, ones_v, semz, sems):
    c = lax.axis_index("c")
    s = lax.axis_index("s")
    b0 = s * _BPS
    zone_base = c * (_LPC * _PLANE) + s * _WPS

    pltpu.sync_copy(idx_hbm.at[pl.ds(b0 * _L, _BPS * _L)], idx_v)
    pltpu.sync_copy(z_hbm, zbuf)

    for j in range(_NZ):
        pltpu.async_copy(
            zbuf, out_hbm.at[pl.ds(zone_base + j * _ZC, _ZC)], semz)

    for g in range(_BPS // 16):
        ones_v[pl.ds(g * 16, 16)] = jnp.full((16,), 1.0, jnp.float32)
    for ll in range(_LPC):
        l = c * _LPC + ll
        for g in range(_BPS // 16):
            bvec = lax.iota(jnp.int32, 16) + (b0 + g * 16)
            iv = plsc.load_gather(idx_v, [(bvec - b0) * _L + l])
            pos_v[ll, pl.ds(g * 16, 16)] = (
                l * _PLANE + (iv >> 3) * 8192 + (bvec >> 7) * 1024
                + (iv & 7) * 128 + (bvec & 127))

    for j in range(_NZ):
        pltpu.make_async_copy(
            zbuf, out_hbm.at[pl.ds(zone_base, _ZC)], semz).wait()
    plsc.subcore_barrier()
    for ll in range(_LPC):
        pltpu.async_copy(ones_v, out_hbm.at[pos_v.at[ll]], sems)
    for ll in range(_LPC):
        pltpu.make_async_copy(
            ones_v, out_hbm.at[pos_v.at[0]], sems).wait()


def kernel(input):
    idx_flat = input.reshape(_B * _L)
    z = jnp.zeros((_ZC,), jnp.float32)
    flat = _onehot_sc(idx_flat, z)
    t5 = flat.reshape(_L, _V // 8, _B // 128, 8, 128)
    return t5.transpose(2, 4, 0, 1, 3).reshape(_B, _L, _V)
